# baseline (speedup 1.0000x reference)
import functools
import numpy as np
import jax
import jax.numpy as jnp
from jax.experimental import pallas as pl
from jax.experimental.pallas import tpu as pltpu

_CFG = [64, 64, 'M', 128, 128, 'M', 256, 256, 256, 'M',
        512, 512, 512, 'M', 512, 512, 512, 'M']

_KROWS = {232: 4, 120: 8, 64: 16, 32: 30, 16: 16}


def _r8(n):
    return ((n + 7) // 8) * 8


def _conv_body(xp_ref, xc_ref, xn_ref, w_ref, b_ref, o_ref, *,
               P, K, H, W, CH):
    xv = jax.lax.broadcasted_iota(jnp.int32, (K, P, 1), 1).reshape(CH, 1)
    yloc = jax.lax.broadcasted_iota(jnp.int32, (K, P, 1), 0).reshape(CH, 1)
    k = pl.program_id(1)
    cat = jnp.concatenate([xp_ref[...][0], xc_ref[...][0], xn_ref[...][0]],
                          axis=0)
    cout = o_ref.shape[2]
    acc = jnp.zeros((CH, cout), jnp.float32)
    for t in range(9):
        off = (t // 3 - 1) * P + (t % 3 - 1)
        lhs = cat[CH + off:2 * CH + off, :]
        acc = acc + jnp.dot(lhs, w_ref[t], preferred_element_type=jnp.float32)
    yg = k * K + yloc
    valid = ((xv >= 1) & (xv <= W) & (yg >= 1) & (yg <= H))
    out = jnp.where(valid, jnp.maximum(acc + b_ref[0:1, :], 0.0), 0.0)
    o_ref[...] = out.astype(o_ref.dtype)[None]


def _conv_layer(xg, w9, b8, H, W):
    B, T, cin = xg.shape
    cout = w9.shape[2]
    P = _r8(W + 2)
    K = _KROWS[P]
    CH = K * P
    nck = -(-(H + 2) // K)
    assert T == (nck + 2) * CH
    body = functools.partial(_conv_body, P=P, K=K, H=H, W=W, CH=CH)
    return pl.pallas_call(
        body,
        grid=(B, nck),
        in_specs=[
            pl.BlockSpec((1, CH, cin), lambda b, k: (b, k, 0)),
            pl.BlockSpec((1, CH, cin), lambda b, k: (b, k + 1, 0)),
            pl.BlockSpec((1, CH, cin), lambda b, k: (b, k + 2, 0)),
            pl.BlockSpec((9, cin, cout), lambda b, k: (0, 0, 0)),
            pl.BlockSpec((8, cout), lambda b, k: (0, 0)),
        ],
        out_specs=pl.BlockSpec((1, CH, cout), lambda b, k: (b, k + 1, 0)),
        out_shape=jax.ShapeDtypeStruct((B, T, cout), jnp.bfloat16),
    )(xg, xg, xg, w9, b8)


_RB = 16


def _pool_body(a_ref, b_ref, c_ref, d_ref, o_ref):
    m = jnp.maximum(jnp.maximum(a_ref[...], b_ref[...]),
                    jnp.maximum(c_ref[...], d_ref[...]))
    o_ref[...] = m


def _pool_layer(h, H, W):
    B, Hp, P, C = h.shape
    Ho, Wo = H // 2, W // 2
    P2 = _r8(Wo + 2)
    nr = -(-(Ho + 2) // _RB)
    rows2 = nr * _RB
    padr = max(0, 2 * rows2 - (Hp + 1))
    padc = max(0, 2 * P2 - (P + 1))
    u = jnp.pad(h, ((0, 0), (1, padr), (1, padc), (0, 0)))
    u = u[:, :2 * rows2, :2 * P2, :]
    a = u[:, 0::2, 0::2, :]
    b = u[:, 0::2, 1::2, :]
    c = u[:, 1::2, 0::2, :]
    d = u[:, 1::2, 1::2, :]
    spec = pl.BlockSpec((1, _RB, P2, C), lambda bb, r: (bb, r, 0, 0))
    return pl.pallas_call(
        _pool_body,
        grid=(B, nr),
        in_specs=[spec, spec, spec, spec],
        out_specs=spec,
        out_shape=jax.ShapeDtypeStruct((B, rows2, P2, C), jnp.bfloat16),
    )(a, b, c, d)


def _rp_body(f_ref, w_ref, wb_ref, cw_ref, cb_ref, rw_ref, rb_ref,
             oc_ref, orr_ref):
    acc = jnp.zeros((112, 512), jnp.float32)
    for t in range(9):
        off = (t // 3 - 1) * 16 + (t % 3 - 1)
        s = 24 + off
        lhs = f_ref[0:1, s:s + 112, :][0]
        acc = acc + jnp.dot(lhs, w_ref[t], preferred_element_type=jnp.float32)
    rp = jnp.maximum(acc + wb_ref[0:1, :], 0.0)
    parts = [rp[16 * y - 14:16 * y - 9, :] for y in range(2, 7)]
    sel = jnp.concatenate(parts + [jnp.zeros((7, 512), jnp.float32)], axis=0)
    selb = sel.astype(jnp.bfloat16)
    logits = jnp.dot(selb, cw_ref[...], preferred_element_type=jnp.float32)
    oc_ref[0:1] = jax.nn.sigmoid(logits + cb_ref[0:1, :])[None]
    reg = jnp.dot(selb, rw_ref[...], preferred_element_type=jnp.float32)
    orr_ref[0:1] = (reg + rb_ref[0:1, :])[None]


def _cls_body(f_ref, w_ref, b_ref, o_ref):
    o_ref[...] = jnp.dot(f_ref[...], w_ref[...],
                         preferred_element_type=jnp.float32) + b_ref[...]


def _bias8(b):
    return jnp.broadcast_to(b.reshape(1, -1), (8, b.shape[0])).astype(jnp.float32)


def _to_guarded(flat, H, W):
    P = _r8(W + 2)
    K = _KROWS[P]
    CH = K * P
    nck = -(-(H + 2) // K)
    tail = CH + (nck * K - (H + 2)) * P
    return jnp.pad(flat, ((0, 0), (CH, tail), (0, 0)))


def kernel(x, params, anchors):
    del anchors
    B = x.shape[0]

    w9s, b8s = [], []
    cin = 3
    for w, b in zip(params['feat_w'], params['feat_b']):
        cout = w.shape[0]
        wt = jnp.transpose(w, (2, 3, 1, 0)).reshape(9, cin, cout)
        if cin == 3:
            wt = jnp.pad(wt, ((0, 0), (0, 5), (0, 0)))
        w9s.append(wt.astype(jnp.bfloat16))
        b8s.append(_bias8(b))
        cin = cout

    H = 224
    P = _r8(H + 2)
    h = jnp.pad(x.transpose(0, 2, 3, 1),
                ((0, 0), (1, 1), (1, P - 1 - H), (0, 5)))
    h = h.astype(jnp.bfloat16)
    spatial = True
    li = 0
    for v in _CFG:
        P = _r8(H + 2)
        K = _KROWS[P]
        CH = K * P
        if v == 'M':
            if not spatial:
                h = h[:, CH:CH + (H + 2) * P, :].reshape(B, H + 2, P, -1)
            h = _pool_layer(h, H, H)
            H //= 2
            h = h[:, :H + 2]
            spatial = True
        else:
            if spatial:
                h = _to_guarded(h.reshape(B, (H + 2) * P, h.shape[3]), H, H)
            h = _conv_layer(h, w9s[li], b8s[li], H, H)
            li += 1
            spatial = False

    feat = h[:, 1:8, 1:8, :]

    fg = jnp.pad(h.reshape(B, 144, 512), ((0, 0), (8, 8), (0, 0)))
    rpw = jnp.transpose(params['rpw_w'], (2, 3, 1, 0)).reshape(9, 512, 512)
    cw = jnp.pad(params['rpc_w'].reshape(3, 512).T, ((0, 0), (0, 13)))
    rw = jnp.pad(params['rpr_w'].reshape(12, 512).T, ((0, 0), (0, 4)))
    cb = _bias8(jnp.pad(params['rpc_b'], (0, 13)))
    rb = _bias8(jnp.pad(params['rpr_b'], (0, 4)))
    oc, orr = pl.pallas_call(
        _rp_body,
        grid=(B,),
        in_specs=[
            pl.BlockSpec((1, 160, 512), lambda b: (b, 0, 0)),
            pl.BlockSpec((9, 512, 512), lambda b: (0, 0, 0)),
            pl.BlockSpec((8, 512), lambda b: (0, 0)),
            pl.BlockSpec((512, 16), lambda b: (0, 0)),
            pl.BlockSpec((8, 16), lambda b: (0, 0)),
            pl.BlockSpec((512, 16), lambda b: (0, 0)),
            pl.BlockSpec((8, 16), lambda b: (0, 0)),
        ],
        out_specs=[pl.BlockSpec((1, 32, 16), lambda b: (b, 0, 0)),
                   pl.BlockSpec((1, 32, 16), lambda b: (b, 0, 0))],
        out_shape=[jax.ShapeDtypeStruct((B, 32, 16), jnp.float32),
                   jax.ShapeDtypeStruct((B, 32, 16), jnp.float32)],
    )(fg, rpw.astype(jnp.bfloat16), _bias8(params['rpw_b']),
      cw.astype(jnp.bfloat16), cb, rw.astype(jnp.bfloat16), rb)

    ffl = feat.transpose(0, 3, 1, 2).reshape(B, 512 * 49)
    ffl = jnp.pad(ffl, ((0, 8 - B), (0, 0))).astype(jnp.bfloat16)
    w2 = params['cls2_w'].T.astype(jnp.bfloat16)
    cls_full = pl.pallas_call(
        _cls_body,
        out_shape=jax.ShapeDtypeStruct((8, 20), jnp.float32),
    )(ffl, w2, _bias8(params['cls2_b']))
    cls_out = cls_full[:B]

    rp_cls = (oc[:, :25, :3].reshape(B, 5, 5, 3)
              .transpose(0, 3, 1, 2).reshape(B, 5, 5, 3))
    rp_reg = (orr[:, :25, :12].reshape(B, 5, 5, 12)
              .transpose(0, 3, 1, 2).reshape(B, 5, 5, 3, 4))
    return (cls_out, rp_cls, rp_reg)


# speedup vs baseline: 1.0487x; 1.0487x over previous
import functools
import numpy as np
import jax
import jax.numpy as jnp
from jax.experimental import pallas as pl
from jax.experimental.pallas import tpu as pltpu

_CFG = [64, 64, 'M', 128, 128, 'M', 256, 256, 256, 'M',
        512, 512, 512, 'M', 512, 512, 512, 'M']

_KROWS = {232: 8, 120: 16, 64: 16, 32: 30, 16: 16}


def _r8(n):
    return ((n + 7) // 8) * 8


def _conv_body(xp_ref, xc_ref, xn_ref, w_ref, b_ref, o_ref, *,
               P, K, H, W, CH):
    xv = jax.lax.broadcasted_iota(jnp.int32, (K, P, 1), 1).reshape(CH, 1)
    yloc = jax.lax.broadcasted_iota(jnp.int32, (K, P, 1), 0).reshape(CH, 1)
    k = pl.program_id(1)
    cat = jnp.concatenate([xp_ref[...][0], xc_ref[...][0], xn_ref[...][0]],
                          axis=0)
    cout = o_ref.shape[2]
    acc = jnp.zeros((CH, cout), jnp.float32)
    for t in range(9):
        off = (t // 3 - 1) * P + (t % 3 - 1)
        lhs = cat[CH + off:2 * CH + off, :]
        acc = acc + jnp.dot(lhs, w_ref[t], preferred_element_type=jnp.float32)
    yg = k * K + yloc
    valid = ((xv >= 1) & (xv <= W) & (yg >= 1) & (yg <= H))
    out = jnp.where(valid, jnp.maximum(acc + b_ref[0:1, :], 0.0), 0.0)
    o_ref[...] = out.astype(o_ref.dtype)[None]


def _conv_layer(xg, w9, b8, H, W):
    B, T, cin = xg.shape
    cout = w9.shape[2]
    P = _r8(W + 2)
    K = _KROWS[P]
    CH = K * P
    nck = -(-(H + 2) // K)
    assert T == (nck + 2) * CH
    body = functools.partial(_conv_body, P=P, K=K, H=H, W=W, CH=CH)
    return pl.pallas_call(
        body,
        grid=(B, nck),
        in_specs=[
            pl.BlockSpec((1, CH, cin), lambda b, k: (b, k, 0)),
            pl.BlockSpec((1, CH, cin), lambda b, k: (b, k + 1, 0)),
            pl.BlockSpec((1, CH, cin), lambda b, k: (b, k + 2, 0)),
            pl.BlockSpec((9, cin, cout), lambda b, k: (0, 0, 0)),
            pl.BlockSpec((8, cout), lambda b, k: (0, 0)),
        ],
        out_specs=pl.BlockSpec((1, CH, cout), lambda b, k: (b, k + 1, 0)),
        out_shape=jax.ShapeDtypeStruct((B, T, cout), jnp.bfloat16),
        compiler_params=pltpu.CompilerParams(
            dimension_semantics=("parallel", "parallel")),
    )(xg, xg, xg, w9, b8)


_RB = 16


def _pool_body(a_ref, b_ref, c_ref, d_ref, o_ref):
    m = jnp.maximum(jnp.maximum(a_ref[...], b_ref[...]),
                    jnp.maximum(c_ref[...], d_ref[...]))
    o_ref[...] = m


def _pool_layer(h, H, W):
    B, Hp, P, C = h.shape
    Ho, Wo = H // 2, W // 2
    P2 = _r8(Wo + 2)
    nr = -(-(Ho + 2) // _RB)
    rows2 = nr * _RB
    padr = max(0, 2 * rows2 - (Hp + 1))
    padc = max(0, 2 * P2 - (P + 1))
    u = jnp.pad(h, ((0, 0), (1, padr), (1, padc), (0, 0)))
    u = u[:, :2 * rows2, :2 * P2, :]
    a = u[:, 0::2, 0::2, :]
    b = u[:, 0::2, 1::2, :]
    c = u[:, 1::2, 0::2, :]
    d = u[:, 1::2, 1::2, :]
    spec = pl.BlockSpec((1, _RB, P2, C), lambda bb, r: (bb, r, 0, 0))
    return pl.pallas_call(
        _pool_body,
        grid=(B, nr),
        in_specs=[spec, spec, spec, spec],
        out_specs=spec,
        out_shape=jax.ShapeDtypeStruct((B, rows2, P2, C), jnp.bfloat16),
        compiler_params=pltpu.CompilerParams(
            dimension_semantics=("parallel", "parallel")),
    )(a, b, c, d)


def _rp_body(f_ref, w_ref, wb_ref, cw_ref, cb_ref, rw_ref, rb_ref,
             oc_ref, orr_ref):
    acc = jnp.zeros((112, 512), jnp.float32)
    for t in range(9):
        off = (t // 3 - 1) * 16 + (t % 3 - 1)
        s = 24 + off
        lhs = f_ref[0:1, s:s + 112, :][0]
        acc = acc + jnp.dot(lhs, w_ref[t], preferred_element_type=jnp.float32)
    rp = jnp.maximum(acc + wb_ref[0:1, :], 0.0)
    parts = [rp[16 * y - 14:16 * y - 9, :] for y in range(2, 7)]
    sel = jnp.concatenate(parts + [jnp.zeros((7, 512), jnp.float32)], axis=0)
    selb = sel.astype(jnp.bfloat16)
    logits = jnp.dot(selb, cw_ref[...], preferred_element_type=jnp.float32)
    oc_ref[0:1] = jax.nn.sigmoid(logits + cb_ref[0:1, :])[None]
    reg = jnp.dot(selb, rw_ref[...], preferred_element_type=jnp.float32)
    orr_ref[0:1] = (reg + rb_ref[0:1, :])[None]


def _cls_body(f_ref, w_ref, b_ref, o_ref):
    o_ref[...] = jnp.dot(f_ref[...], w_ref[...],
                         preferred_element_type=jnp.float32) + b_ref[...]


def _bias8(b):
    return jnp.broadcast_to(b.reshape(1, -1), (8, b.shape[0])).astype(jnp.float32)


def _to_guarded(flat, H, W):
    P = _r8(W + 2)
    K = _KROWS[P]
    CH = K * P
    nck = -(-(H + 2) // K)
    tail = CH + (nck * K - (H + 2)) * P
    return jnp.pad(flat, ((0, 0), (CH, tail), (0, 0)))


def kernel(x, params, anchors):
    del anchors
    B = x.shape[0]

    w9s, b8s = [], []
    cin = 3
    for w, b in zip(params['feat_w'], params['feat_b']):
        cout = w.shape[0]
        wt = jnp.transpose(w, (2, 3, 1, 0)).reshape(9, cin, cout)
        if cin == 3:
            wt = jnp.pad(wt, ((0, 0), (0, 5), (0, 0)))
        w9s.append(wt.astype(jnp.bfloat16))
        b8s.append(_bias8(b))
        cin = cout

    H = 224
    P = _r8(H + 2)
    h = jnp.pad(x.transpose(0, 2, 3, 1),
                ((0, 0), (1, 1), (1, P - 1 - H), (0, 5)))
    h = h.astype(jnp.bfloat16)
    spatial = True
    li = 0
    for v in _CFG:
        P = _r8(H + 2)
        K = _KROWS[P]
        CH = K * P
        if v == 'M':
            if not spatial:
                h = h[:, CH:CH + (H + 2) * P, :].reshape(B, H + 2, P, -1)
            h = _pool_layer(h, H, H)
            H //= 2
            h = h[:, :H + 2]
            spatial = True
        else:
            if spatial:
                h = _to_guarded(h.reshape(B, (H + 2) * P, h.shape[3]), H, H)
            h = _conv_layer(h, w9s[li], b8s[li], H, H)
            li += 1
            spatial = False

    feat = h[:, 1:8, 1:8, :]

    fg = jnp.pad(h.reshape(B, 144, 512), ((0, 0), (8, 8), (0, 0)))
    rpw = jnp.transpose(params['rpw_w'], (2, 3, 1, 0)).reshape(9, 512, 512)
    cw = jnp.pad(params['rpc_w'].reshape(3, 512).T, ((0, 0), (0, 13)))
    rw = jnp.pad(params['rpr_w'].reshape(12, 512).T, ((0, 0), (0, 4)))
    cb = _bias8(jnp.pad(params['rpc_b'], (0, 13)))
    rb = _bias8(jnp.pad(params['rpr_b'], (0, 4)))
    oc, orr = pl.pallas_call(
        _rp_body,
        grid=(B,),
        in_specs=[
            pl.BlockSpec((1, 160, 512), lambda b: (b, 0, 0)),
            pl.BlockSpec((9, 512, 512), lambda b: (0, 0, 0)),
            pl.BlockSpec((8, 512), lambda b: (0, 0)),
            pl.BlockSpec((512, 16), lambda b: (0, 0)),
            pl.BlockSpec((8, 16), lambda b: (0, 0)),
            pl.BlockSpec((512, 16), lambda b: (0, 0)),
            pl.BlockSpec((8, 16), lambda b: (0, 0)),
        ],
        out_specs=[pl.BlockSpec((1, 32, 16), lambda b: (b, 0, 0)),
                   pl.BlockSpec((1, 32, 16), lambda b: (b, 0, 0))],
        out_shape=[jax.ShapeDtypeStruct((B, 32, 16), jnp.float32),
                   jax.ShapeDtypeStruct((B, 32, 16), jnp.float32)],
    )(fg, rpw.astype(jnp.bfloat16), _bias8(params['rpw_b']),
      cw.astype(jnp.bfloat16), cb, rw.astype(jnp.bfloat16), rb)

    ffl = feat.transpose(0, 3, 1, 2).reshape(B, 512 * 49)
    ffl = jnp.pad(ffl, ((0, 8 - B), (0, 0))).astype(jnp.bfloat16)
    w2 = params['cls2_w'].T.astype(jnp.bfloat16)
    cls_full = pl.pallas_call(
        _cls_body,
        out_shape=jax.ShapeDtypeStruct((8, 20), jnp.float32),
    )(ffl, w2, _bias8(params['cls2_b']))
    cls_out = cls_full[:B]

    rp_cls = (oc[:, :25, :3].reshape(B, 5, 5, 3)
              .transpose(0, 3, 1, 2).reshape(B, 5, 5, 3))
    rp_reg = (orr[:, :25, :12].reshape(B, 5, 5, 12)
              .transpose(0, 3, 1, 2).reshape(B, 5, 5, 3, 4))
    return (cls_out, rp_cls, rp_reg)
